# baseline (speedup 1.0000x reference)
import functools

import jax
import jax.numpy as jnp
from jax import lax
from jax.experimental import pallas as pl
from jax.experimental.pallas import tpu as pltpu
from jax.experimental.pallas import tpu_sc as plsc

N = 50000
E = 800000
D = 48
DE = 9
L = 4
GS = 16
NH = 25000

NC = 2
NS = 16

EB = 128
E_P = 800768
EPT = E_P // NS
NB_E = EPT // EB
NPAIR_E = (NB_E + 1) // 2

BLK = 25
PADR = 48
NBR = N // BLK
N_P = N + BLK * PADR
NB_N = N_P // EB
EBLK = 6
NHB = 4168
NHP = 25008
NBF = (N + EB - 1) // EB
FTAIL = N - (NBF - 1) * EB
FPAD = NBF * EB

f32 = jnp.float32

_USE_SC_EDGE = True
_USE_SC_SEG = True
_USE_SC_FIN = True


def _z(r, c):
    return jnp.zeros((r, c), f32)


def _split_pad(x):
    r = x.shape[0]
    return (jnp.concatenate([x[:, :24], _z(r, 8)], axis=1),
            jnp.concatenate([x[:, 24:], _z(r, 8)], axis=1))


def _t0_body(embed_ref, wx_ref, ta_ref, tb_ref):
    t = embed_ref[...] @ wx_ref[...]
    ta_ref[...], tb_ref[...] = _split_pad(t)


def _ew_body(ea_ref, we_ref, oa_ref, ob_ref):
    r = ea_ref[...] @ we_ref[...]
    oa_ref[...], ob_ref[...] = _split_pad(r)


def _c_body(nma_ref, nmb_ref, hv_ref, wg_ref, wn_ref,
            xaa_ref, xab_ref, xwa_ref, xwb_ref):
    nm = jnp.concatenate([nma_ref[...][:, :24], nmb_ref[...][:, :24]], axis=1)
    g = nm @ wg_ref[...]
    xa = jnp.concatenate([jax.nn.sigmoid(g[:, :GS]), jnp.tanh(g[:, GS:])], axis=1)
    hv = hv_ref[...]
    zp = _z(PADR, 32)
    xaa_ref[...] = jnp.concatenate(
        [jnp.concatenate([xa[:, :24], hv, _z(NBR, 7)], axis=1), zp], axis=0)
    xab_ref[...] = jnp.concatenate(
        [jnp.concatenate([xa[:, 24:], _z(NBR, 8)], axis=1), zp], axis=0)
    xw = xa @ wn_ref[...]
    xwa_ref[...], xwb_ref[...] = _split_pad(xw)


def _c3_body(nma_ref, nmb_ref, hv_ref, wg_ref, wh_ref, bh_ref,
             xaa_ref, xab_ref, ao_ref):
    nm = jnp.concatenate([nma_ref[...][:, :24], nmb_ref[...][:, :24]], axis=1)
    g = nm @ wg_ref[...]
    xa = jnp.concatenate([jax.nn.sigmoid(g[:, :GS]), jnp.tanh(g[:, GS:])], axis=1)
    hv = hv_ref[...]
    zp = _z(PADR, 32)
    xaa_ref[...] = jnp.concatenate(
        [jnp.concatenate([xa[:, :24], hv, _z(NBR, 7)], axis=1), zp], axis=0)
    xab_ref[...] = jnp.concatenate(
        [jnp.concatenate([xa[:, 24:], _z(NBR, 8)], axis=1), zp], axis=0)
    ao_ref[...] = xa @ wh_ref[...] + bh_ref[...]


def _e_body(sma_ref, smb_ref, wha_ref, whb_ref, wn_ref, twa_ref, twb_ref):
    sa = sma_ref[...]
    sums = jnp.concatenate([sa[:, :24], smb_ref[...][:, :24]], axis=1)
    cnt = sa[:, 24:25]
    xh = sums / jnp.maximum(cnt, 1.0)
    tp = (xh @ wha_ref[...]) * (xh @ whb_ref[...])
    tw = tp @ wn_ref[...]
    twa_ref[...], twb_ref[...] = _split_pad(tw)


def _e3_body(sma_ref, smb_ref, wha_ref, whb_ref, wh_ref, bh_ref, bo_ref):
    sa = sma_ref[...]
    sums = jnp.concatenate([sa[:, :24], smb_ref[...][:, :24]], axis=1)
    cnt = sa[:, 24:25]
    xh = sums / jnp.maximum(cnt, 1.0)
    tp = (xh @ wha_ref[...]) * (xh @ whb_ref[...])
    bo_ref[...] = tp @ wh_ref[...] + bh_ref[...]


def _tc_t0(embed, wx0):
    return pl.pallas_call(
        _t0_body,
        out_shape=[jax.ShapeDtypeStruct((128, 32), f32)] * 2,
    )(embed, wx0)


def _tc_ew(ea_p, we):
    nblk = E_P // 2048
    return pl.pallas_call(
        _ew_body,
        grid=(nblk,),
        in_specs=[pl.BlockSpec((2048, DE), lambda i: (i, 0)),
                  pl.BlockSpec((DE, D), lambda i: (0, 0))],
        out_specs=[pl.BlockSpec((2048, 32), lambda i: (i, 0))] * 2,
        out_shape=[jax.ShapeDtypeStruct((E_P, 32), f32)] * 2,
    )(ea_p, we)


def _tc_c(nma, nmb, hv, wg, wn):
    return pl.pallas_call(
        _c_body,
        grid=(BLK,),
        in_specs=[pl.BlockSpec((NBR, 32), lambda i: (i, 0)),
                  pl.BlockSpec((NBR, 32), lambda i: (i, 0)),
                  pl.BlockSpec((NBR, 1), lambda i: (i, 0)),
                  pl.BlockSpec((D, D), lambda i: (0, 0)),
                  pl.BlockSpec((D, D), lambda i: (0, 0))],
        out_specs=[pl.BlockSpec((NBR + PADR, 32), lambda i: (i, 0)),
                   pl.BlockSpec((NBR + PADR, 32), lambda i: (i, 0)),
                   pl.BlockSpec((NBR, 32), lambda i: (i, 0)),
                   pl.BlockSpec((NBR, 32), lambda i: (i, 0))],
        out_shape=[jax.ShapeDtypeStruct((N_P, 32), f32),
                   jax.ShapeDtypeStruct((N_P, 32), f32),
                   jax.ShapeDtypeStruct((N, 32), f32),
                   jax.ShapeDtypeStruct((N, 32), f32)],
    )(nma, nmb, hv, wg, wn)


def _tc_c3(nma, nmb, hv, wg, wh, bh):
    return pl.pallas_call(
        _c3_body,
        grid=(BLK,),
        in_specs=[pl.BlockSpec((NBR, 32), lambda i: (i, 0)),
                  pl.BlockSpec((NBR, 32), lambda i: (i, 0)),
                  pl.BlockSpec((NBR, 1), lambda i: (i, 0)),
                  pl.BlockSpec((D, D), lambda i: (0, 0)),
                  pl.BlockSpec((D, 1), lambda i: (0, 0)),
                  pl.BlockSpec((1, 1), lambda i: (0, 0))],
        out_specs=[pl.BlockSpec((NBR + PADR, 32), lambda i: (i, 0)),
                   pl.BlockSpec((NBR + PADR, 32), lambda i: (i, 0)),
                   pl.BlockSpec((NBR, 1), lambda i: (i, 0))],
        out_shape=[jax.ShapeDtypeStruct((N_P, 32), f32),
                   jax.ShapeDtypeStruct((N_P, 32), f32),
                   jax.ShapeDtypeStruct((N, 1), f32)],
    )(nma, nmb, hv, wg, wh, bh)


def _tc_e(sma, smb, wha, whb, wn):
    return pl.pallas_call(
        _e_body,
        grid=(EBLK,),
        in_specs=[pl.BlockSpec((NHB, 32), lambda i: (i, 0)),
                  pl.BlockSpec((NHB, 32), lambda i: (i, 0)),
                  pl.BlockSpec((D, D), lambda i: (0, 0)),
                  pl.BlockSpec((D, D), lambda i: (0, 0)),
                  pl.BlockSpec((D, D), lambda i: (0, 0))],
        out_specs=[pl.BlockSpec((NHB, 32), lambda i: (i, 0))] * 2,
        out_shape=[jax.ShapeDtypeStruct((NHP, 32), f32)] * 2,
    )(sma, smb, wha, whb, wn)


def _tc_e3(sma, smb, wha, whb, wh, bh):
    return pl.pallas_call(
        _e3_body,
        grid=(EBLK,),
        in_specs=[pl.BlockSpec((NHB, 32), lambda i: (i, 0)),
                  pl.BlockSpec((NHB, 32), lambda i: (i, 0)),
                  pl.BlockSpec((D, D), lambda i: (0, 0)),
                  pl.BlockSpec((D, D), lambda i: (0, 0)),
                  pl.BlockSpec((D, 1), lambda i: (0, 0)),
                  pl.BlockSpec((1, 1), lambda i: (0, 0))],
        out_specs=pl.BlockSpec((NHB, 1), lambda i: (i, 0)),
        out_shape=jax.ShapeDtypeStruct((NHP, 1), f32),
    )(sma, smb, wha, whb, wh, bh)


def _sc_edge_body(ta_ref, tb_ref, ewa_ref, ewb_ref, eidx_ref, dst_ref,
                  nma_ref, nmb_ref,
                  acc, gi0, gi1, di0, di1, er0, er1, rw0, rw1, zb,
                  si0, si1, sd0, sd1, se0, se1, sg0, sg1, ss0, ss1):
    c = lax.axis_index("c")
    s = lax.axis_index("s")

    def _zrow(i, _):
        zb[i, pl.ds(0, 16)] = jnp.zeros((16,), f32)
        zb[i, pl.ds(16, 16)] = jnp.zeros((16,), f32)
        return 0
    lax.fori_loop(0, 125, _zrow, 0)

    def _zcp(j, _):
        pltpu.sync_copy(zb, acc.at[pl.ds(s * 3125 + j * 125, 125), :])
        return 0
    lax.fori_loop(0, 25, _zcp, 0)
    plsc.subcore_barrier()

    def _run(t_ref, ew_ref):
        base = s * EPT
        gi = (gi0, gi1)
        di = (di0, di1)
        er = (er0, er1)
        rw = (rw0, rw1)
        si = (si0, si1)
        sd = (sd0, sd1)
        se = (se0, se1)
        sg = (sg0, sg1)
        ss = (ss0, ss1)

        def start_inputs(b, p):
            e0 = base + b * EB
            pltpu.async_copy(eidx_ref.at[pl.ds(e0, EB)], gi[p], si[p])
            pltpu.async_copy(dst_ref.at[pl.ds(e0, EB)], di[p], sd[p])
            pltpu.async_copy(ew_ref.at[pl.ds(e0, EB), :], er[p], se[p])

        def wait_idx(p):
            pltpu.make_async_copy(eidx_ref.at[pl.ds(0, EB)], gi[p], si[p]).wait()

        def wait_dst(p):
            pltpu.make_async_copy(dst_ref.at[pl.ds(0, EB)], di[p], sd[p]).wait()

        def wait_ew(p):
            pltpu.make_async_copy(ew_ref.at[pl.ds(0, EB), :], er[p], se[p]).wait()

        def start_gather(p):
            pltpu.async_copy(t_ref.at[gi[p]], rw[p], sg[p])

        def wait_gather(p):
            pltpu.make_async_copy(t_ref.at[gi[p]], rw[p], sg[p]).wait()

        def start_scatter(p):
            pltpu.async_copy(rw[p], acc.at[di[p]], ss[p], add=True)

        def wait_scatter(p):
            pltpu.make_async_copy(rw[p], acc.at[di[p]], ss[p]).wait()

        def multiply(p):
            rb, eb = rw[p], er[p]

            def _m(i, _):
                rb[i, pl.ds(0, 16)] = rb[i, pl.ds(0, 16)] * eb[i, pl.ds(0, 16)]
                rb[i, pl.ds(16, 16)] = rb[i, pl.ds(16, 16)] * eb[i, pl.ds(16, 16)]
                return 0
            lax.fori_loop(0, EB, _m, 0)

        start_inputs(0, 0)
        wait_idx(0)
        start_gather(0)

        def pair(t, _):
            b1 = 2 * t + 1
            b2 = 2 * t + 2

            @pl.when(b1 < NB_E)
            def _():
                @pl.when(t > 0)
                def _():
                    wait_scatter(1)
                start_inputs(b1, 1)

            wait_ew(0)
            wait_dst(0)
            wait_gather(0)
            multiply(0)
            start_scatter(0)

            @pl.when(b1 < NB_E)
            def _():
                @pl.when(b2 < NB_E)
                def _():
                    wait_scatter(0)
                    start_inputs(b2, 0)
                wait_idx(1)
                start_gather(1)
                wait_ew(1)
                wait_dst(1)
                wait_gather(1)
                multiply(1)
                start_scatter(1)

                @pl.when(b2 < NB_E)
                def _():
                    wait_idx(0)
                    start_gather(0)
            return 0

        lax.fori_loop(0, NPAIR_E, pair, 0)
        wait_scatter(0)
        wait_scatter(1)

    @pl.when(c == 0)
    def _():
        _run(ta_ref, ewa_ref)

    @pl.when(c == 1)
    def _():
        _run(tb_ref, ewb_ref)

    plsc.subcore_barrier()

    def _wout(nm_ref):
        @pl.when(s < NS - 1)
        def _():
            pltpu.sync_copy(acc.at[pl.ds(s * 3128, 3128), :],
                            nm_ref.at[pl.ds(s * 3128, 3128), :])

        @pl.when(s == NS - 1)
        def _():
            pltpu.sync_copy(acc.at[pl.ds(15 * 3128, 3080), :],
                            nm_ref.at[pl.ds(15 * 3128, 3080), :])

    @pl.when(c == 0)
    def _():
        _wout(nma_ref)

    @pl.when(c == 1)
    def _():
        _wout(nmb_ref)


def _sc_edge(ta, tb, ewa, ewb, eidx, dst):
    mesh = plsc.VectorSubcoreMesh(core_axis_name="c", subcore_axis_name="s")
    fn = pl.kernel(
        _sc_edge_body,
        out_type=[jax.ShapeDtypeStruct((N, 32), f32)] * 2,
        mesh=mesh,
        compiler_params=pltpu.CompilerParams(use_tc_tiling_on_sc=False, needs_layout_passes=False),
        scratch_types=[
            pltpu.VMEM_SHARED((N, 32), f32),
            pltpu.VMEM((EB,), jnp.int32), pltpu.VMEM((EB,), jnp.int32),
            pltpu.VMEM((EB,), jnp.int32), pltpu.VMEM((EB,), jnp.int32),
            pltpu.VMEM((EB, 32), f32), pltpu.VMEM((EB, 32), f32),
            pltpu.VMEM((EB, 32), f32), pltpu.VMEM((EB, 32), f32),
            pltpu.VMEM((125, 32), f32),
        ] + [pltpu.SemaphoreType.DMA] * 10,
    )
    return fn(ta, tb, ewa, ewb, eidx, dst)


def _sc_seg_body(xaa_ref, xab_ref, seg_ref, sma_ref, smb_ref,
                 acc, si_0, si_1, xr0, xr1, zb,
                 s_i0, s_i1, s_x0, s_x1, s_s0, s_s1):
    c = lax.axis_index("c")
    s = lax.axis_index("s")

    def _zrow(i, _):
        zb[i, pl.ds(0, 16)] = jnp.zeros((16,), f32)
        zb[i, pl.ds(16, 16)] = jnp.zeros((16,), f32)
        return 0
    lax.fori_loop(0, 125, _zrow, 0)

    def _zcp(j, _):
        pltpu.sync_copy(zb.at[pl.ds(0, 121), :], acc.at[pl.ds(s * 1563 + j * 121, 121), :])
        return 0
    lax.fori_loop(0, 12, _zcp, 0)

    pltpu.sync_copy(zb.at[pl.ds(0, 111), :], acc.at[pl.ds(s * 1563 + 1452, 111), :])
    plsc.subcore_barrier()

    def _run(xa_ref):
        si = (si_0, si_1)
        xr = (xr0, xr1)
        ssi = (s_i0, s_i1)
        ssx = (s_x0, s_x1)
        sss = (s_s0, s_s1)

        def start_inputs(b, p):
            r0 = b * EB
            pltpu.async_copy(seg_ref.at[pl.ds(r0, EB)], si[p], ssi[p])
            pltpu.async_copy(xa_ref.at[pl.ds(r0, EB), :], xr[p], ssx[p])

        def wait_inputs(p):
            pltpu.make_async_copy(seg_ref.at[pl.ds(0, EB)], si[p], ssi[p]).wait()
            pltpu.make_async_copy(xa_ref.at[pl.ds(0, EB), :], xr[p], ssx[p]).wait()

        def start_scatter(p):
            pltpu.async_copy(xr[p], acc.at[si[p]], sss[p], add=True)

        def wait_scatter(p):
            pltpu.make_async_copy(xr[p], acc.at[si[p]], sss[p]).wait()

        @pl.when(s < NB_N)
        def _():
            start_inputs(s, 0)

        def pair(t, _):
            j0 = 2 * t
            j1 = 2 * t + 1
            b0 = s + 16 * j0
            b1 = s + 16 * j1
            b2 = s + 16 * (j1 + 1)

            @pl.when(b0 < NB_N)
            def _():
                @pl.when(b1 < NB_N)
                def _():
                    @pl.when(t > 0)
                    def _():
                        wait_scatter(1)
                    start_inputs(b1, 1)
                wait_inputs(0)
                start_scatter(0)

            @pl.when(b1 < NB_N)
            def _():
                @pl.when(b2 < NB_N)
                def _():
                    wait_scatter(0)
                    start_inputs(b2, 0)
                wait_inputs(1)
                start_scatter(1)
            return 0

        lax.fori_loop(0, 13, pair, 0)

        @pl.when(s < NB_N)
        def _():
            wait_scatter(0)

        @pl.when(s + 16 < NB_N)
        def _():
            wait_scatter(1)

    @pl.when(c == 0)
    def _():
        _run(xaa_ref)

    @pl.when(c == 1)
    def _():
        _run(xab_ref)

    plsc.subcore_barrier()

    def _wout(sm_ref):
        @pl.when(s < NS - 1)
        def _():
            pltpu.sync_copy(acc.at[pl.ds(s * 1568, 1568), :],
                            sm_ref.at[pl.ds(s * 1568, 1568), :])

        @pl.when(s == NS - 1)
        def _():
            pltpu.sync_copy(acc.at[pl.ds(15 * 1568, 1488), :],
                            sm_ref.at[pl.ds(15 * 1568, 1488), :])

    @pl.when(c == 0)
    def _():
        _wout(sma_ref)

    @pl.when(c == 1)
    def _():
        _wout(smb_ref)


def _sc_seg(xaa, xab, seg_p):
    mesh = plsc.VectorSubcoreMesh(core_axis_name="c", subcore_axis_name="s")
    fn = pl.kernel(
        _sc_seg_body,
        out_type=[jax.ShapeDtypeStruct((NHP, 32), f32)] * 2,
        mesh=mesh,
        compiler_params=pltpu.CompilerParams(use_tc_tiling_on_sc=False, needs_layout_passes=False),
        scratch_types=[
            pltpu.VMEM_SHARED((NHP, 32), f32),
            pltpu.VMEM((EB,), jnp.int32), pltpu.VMEM((EB,), jnp.int32),
            pltpu.VMEM((EB, 32), f32), pltpu.VMEM((EB, 32), f32),
            pltpu.VMEM((125, 32), f32),
        ] + [pltpu.SemaphoreType.DMA] * 6,
    )
    return fn(xaa, xab, seg_p)


def _sc_fin_body(tf_ref, fidx_ref, out_ref, tfv, fv, ov, s_t, s_f):
    c = lax.axis_index("c")
    s = lax.axis_index("s")
    wid = c * NS + s
    pltpu.async_copy(tf_ref, tfv, s_t).wait()

    def step(j, _):
        b = wid + 32 * j

        @pl.when(b < NBF)
        def _():
            pltpu.async_copy(fidx_ref.at[pl.ds(b * EB, EB)], fv, s_f).wait()
            for k in range(8):
                idx16 = fv[pl.ds(k * 16, 16)]
                ov[pl.ds(k * 16, 16)] = plsc.load_gather(tfv, [idx16])

            @pl.when(b < NBF - 1)
            def _():
                pltpu.sync_copy(ov, out_ref.at[pl.ds(b * EB, EB)])

            @pl.when(b == NBF - 1)
            def _():
                pltpu.sync_copy(ov.at[pl.ds(0, FTAIL)],
                                out_ref.at[pl.ds(b * EB, FTAIL)])
        return 0

    lax.fori_loop(0, 13, step, 0)


def _sc_fin(tf, fidx_p):
    mesh = plsc.VectorSubcoreMesh(core_axis_name="c", subcore_axis_name="s")
    fn = pl.kernel(
        _sc_fin_body,
        out_type=jax.ShapeDtypeStruct((N,), f32),
        mesh=mesh,
        compiler_params=pltpu.CompilerParams(use_tc_tiling_on_sc=False, needs_layout_passes=False),
        scratch_types=[
            pltpu.VMEM((N + NH,), f32),
            pltpu.VMEM((EB,), jnp.int32),
            pltpu.VMEM((EB,), f32),
            pltpu.SemaphoreType.DMA, pltpu.SemaphoreType.DMA,
        ],
    )
    return fn(tf, fidx_p)


def kernel(node_idx, edge_index_no, edge_attr_no, z, canonical, embed,
           Wx, We, Wg, Wha, Whb, W_head, b_head):
    src = edge_index_no[0]
    dst = edge_index_no[1]
    heavy = z > 1
    seg = jnp.where(heavy, canonical, NH).astype(jnp.int32)
    segs = [jnp.concatenate([seg[i * NBR:(i + 1) * NBR],
                             jnp.full((PADR,), NH, jnp.int32)]) for i in range(BLK)]
    seg_p = jnp.concatenate(segs)
    hv = heavy.astype(f32)[:, None]

    ezpad = jnp.zeros((E_P - E,), jnp.int32)
    eidx0_p = jnp.concatenate([jnp.take(node_idx, src).astype(jnp.int32), ezpad])
    heavy_src = jnp.take(heavy, src)
    can_src = jnp.take(canonical, src)
    eidx_p = jnp.concatenate([
        jnp.where(heavy_src, N + can_src, src).astype(jnp.int32), ezpad])
    dst_p = jnp.concatenate([dst.astype(jnp.int32), ezpad])
    ea_p = jnp.concatenate([edge_attr_no, jnp.zeros((E_P - E, DE), f32)])

    fidx = jnp.where(heavy, N + canonical, jnp.arange(N)).astype(jnp.int32)
    fidx_p = jnp.concatenate([fidx, jnp.zeros((FPAD - N,), jnp.int32)])

    def _jnp_edge(ta, tb, ewa, ewb, eidx_l, dst_p):
        t48 = jnp.concatenate([ta[:, :24], tb[:, :24]], axis=1)
        ew48 = jnp.concatenate([ewa[:, :24], ewb[:, :24]], axis=1)
        nm48 = jax.ops.segment_sum(jnp.take(t48, eidx_l, axis=0) * ew48,
                                   dst_p, num_segments=N)
        zz = jnp.zeros((N, 8), f32)
        return (jnp.concatenate([nm48[:, :24], zz], axis=1),
                jnp.concatenate([nm48[:, 24:], zz], axis=1))

    def _jnp_seg(xaa, xab, seg_p):
        return (jax.ops.segment_sum(xaa, seg_p, num_segments=NHP),
                jax.ops.segment_sum(xab, seg_p, num_segments=NHP))

    def _jnp_fin(tf, fidx_p):
        return jnp.take(tf, fidx_p[:N])

    ta, tb = _tc_t0(embed, Wx[0])
    eidx_l = eidx0_p
    aout = bout = None
    for l in range(L):
        ewa, ewb = _tc_ew(ea_p, We[l])
        if _USE_SC_EDGE:
            nma, nmb = _sc_edge(ta, tb, ewa, ewb, eidx_l, dst_p)
        else:
            nma, nmb = _jnp_edge(ta, tb, ewa, ewb, eidx_l, dst_p)
        if l < L - 1:
            xaa, xab, xwa, xwb = _tc_c(nma, nmb, hv, Wg[l], Wx[l + 1])
            sma, smb = (_sc_seg if _USE_SC_SEG else _jnp_seg)(xaa, xab, seg_p)
            twa, twb = _tc_e(sma, smb, Wha[l], Whb[l], Wx[l + 1])
            ta = jnp.concatenate([xwa, twa])
            tb = jnp.concatenate([xwb, twb])
            eidx_l = eidx_p
        else:
            bh = b_head.reshape(1, 1)
            xaa, xab, aout = _tc_c3(nma, nmb, hv, Wg[l], W_head, bh)
            sma, smb = (_sc_seg if _USE_SC_SEG else _jnp_seg)(xaa, xab, seg_p)
            bout = _tc_e3(sma, smb, Wha[l], Whb[l], W_head, bh)

    tf = jnp.concatenate([aout[:, 0], bout[:NH, 0]])
    out = (_sc_fin if _USE_SC_FIN else _jnp_fin)(tf, fidx_p)
    return out[:, None]


# speedup vs baseline: 2.9768x; 2.9768x over previous
import functools

import jax
import jax.numpy as jnp
from jax import lax
from jax.experimental import pallas as pl
from jax.experimental.pallas import tpu as pltpu
from jax.experimental.pallas import tpu_sc as plsc

N = 50000
E = 800000
D = 48
DE = 9
L = 4
GS = 16
NH = 25000

NC = 2
NS = 16

EB = 128
E_P = 800768
EPT = E_P // NS
NB_E = EPT // EB
NPAIR_E = (NB_E + 1) // 2

BLK = 25
PADR = 48
NBR = N // BLK
N_P = N + BLK * PADR
NB_N = N_P // EB
EBLK = 6
NHB = 4168
NHP = 25008
NBF = (N + EB - 1) // EB
FTAIL = N - (NBF - 1) * EB
FPAD = NBF * EB

f32 = jnp.float32

_USE_SC_EDGE = True
_USE_SC_SEG = True
_USE_SC_FIN = True


def _z(r, c):
    return jnp.zeros((r, c), f32)


def _split_pad(x):
    r = x.shape[0]
    return (jnp.concatenate([x[:, :24], _z(r, 8)], axis=1),
            jnp.concatenate([x[:, 24:], _z(r, 8)], axis=1))


def _t0_body(embed_ref, wx_ref, ta_ref, tb_ref):
    t = embed_ref[...] @ wx_ref[...]
    ta_ref[...], tb_ref[...] = _split_pad(t)


def _ew_body(ea_ref, we_ref, oa_ref, ob_ref):
    r = ea_ref[...] @ we_ref[...]
    oa_ref[...], ob_ref[...] = _split_pad(r)


def _c_body(nma_ref, nmb_ref, hv_ref, wg_ref, wn_ref,
            xaa_ref, xab_ref, xwa_ref, xwb_ref):
    nm = jnp.concatenate([nma_ref[...][:, :24], nmb_ref[...][:, :24]], axis=1)
    g = nm @ wg_ref[...]
    xa = jnp.concatenate([jax.nn.sigmoid(g[:, :GS]), jnp.tanh(g[:, GS:])], axis=1)
    hv = hv_ref[...]
    zp = _z(PADR, 32)
    xaa_ref[...] = jnp.concatenate(
        [jnp.concatenate([xa[:, :24], hv, _z(NBR, 7)], axis=1), zp], axis=0)
    xab_ref[...] = jnp.concatenate(
        [jnp.concatenate([xa[:, 24:], _z(NBR, 8)], axis=1), zp], axis=0)
    xw = xa @ wn_ref[...]
    xwa_ref[...], xwb_ref[...] = _split_pad(xw)


def _c3_body(nma_ref, nmb_ref, hv_ref, wg_ref, wh_ref, bh_ref,
             xaa_ref, xab_ref, ao_ref):
    nm = jnp.concatenate([nma_ref[...][:, :24], nmb_ref[...][:, :24]], axis=1)
    g = nm @ wg_ref[...]
    xa = jnp.concatenate([jax.nn.sigmoid(g[:, :GS]), jnp.tanh(g[:, GS:])], axis=1)
    hv = hv_ref[...]
    zp = _z(PADR, 32)
    xaa_ref[...] = jnp.concatenate(
        [jnp.concatenate([xa[:, :24], hv, _z(NBR, 7)], axis=1), zp], axis=0)
    xab_ref[...] = jnp.concatenate(
        [jnp.concatenate([xa[:, 24:], _z(NBR, 8)], axis=1), zp], axis=0)
    ao_ref[...] = xa @ wh_ref[...] + bh_ref[...]


def _e_body(sma_ref, smb_ref, wha_ref, whb_ref, wn_ref, twa_ref, twb_ref):
    sa = sma_ref[...]
    sums = jnp.concatenate([sa[:, :24], smb_ref[...][:, :24]], axis=1)
    cnt = sa[:, 24:25]
    xh = sums / jnp.maximum(cnt, 1.0)
    tp = (xh @ wha_ref[...]) * (xh @ whb_ref[...])
    tw = tp @ wn_ref[...]
    twa_ref[...], twb_ref[...] = _split_pad(tw)


def _e3_body(sma_ref, smb_ref, wha_ref, whb_ref, wh_ref, bh_ref, bo_ref):
    sa = sma_ref[...]
    sums = jnp.concatenate([sa[:, :24], smb_ref[...][:, :24]], axis=1)
    cnt = sa[:, 24:25]
    xh = sums / jnp.maximum(cnt, 1.0)
    tp = (xh @ wha_ref[...]) * (xh @ whb_ref[...])
    bo_ref[...] = tp @ wh_ref[...] + bh_ref[...]


def _tc_t0(embed, wx0):
    return pl.pallas_call(
        _t0_body,
        out_shape=[jax.ShapeDtypeStruct((128, 32), f32)] * 2,
    )(embed, wx0)


def _tc_ew(ea_p, we):
    nblk = E_P // 2048
    return pl.pallas_call(
        _ew_body,
        grid=(nblk,),
        in_specs=[pl.BlockSpec((2048, DE), lambda i: (i, 0)),
                  pl.BlockSpec((DE, D), lambda i: (0, 0))],
        out_specs=[pl.BlockSpec((2048, 32), lambda i: (i, 0))] * 2,
        out_shape=[jax.ShapeDtypeStruct((E_P, 32), f32)] * 2,
    )(ea_p, we)


def _tc_c(nma, nmb, hv, wg, wn):
    return pl.pallas_call(
        _c_body,
        grid=(BLK,),
        in_specs=[pl.BlockSpec((NBR, 32), lambda i: (i, 0)),
                  pl.BlockSpec((NBR, 32), lambda i: (i, 0)),
                  pl.BlockSpec((NBR, 1), lambda i: (i, 0)),
                  pl.BlockSpec((D, D), lambda i: (0, 0)),
                  pl.BlockSpec((D, D), lambda i: (0, 0))],
        out_specs=[pl.BlockSpec((NBR + PADR, 32), lambda i: (i, 0)),
                   pl.BlockSpec((NBR + PADR, 32), lambda i: (i, 0)),
                   pl.BlockSpec((NBR, 32), lambda i: (i, 0)),
                   pl.BlockSpec((NBR, 32), lambda i: (i, 0))],
        out_shape=[jax.ShapeDtypeStruct((N_P, 32), f32),
                   jax.ShapeDtypeStruct((N_P, 32), f32),
                   jax.ShapeDtypeStruct((N, 32), f32),
                   jax.ShapeDtypeStruct((N, 32), f32)],
    )(nma, nmb, hv, wg, wn)


def _tc_c3(nma, nmb, hv, wg, wh, bh):
    return pl.pallas_call(
        _c3_body,
        grid=(BLK,),
        in_specs=[pl.BlockSpec((NBR, 32), lambda i: (i, 0)),
                  pl.BlockSpec((NBR, 32), lambda i: (i, 0)),
                  pl.BlockSpec((NBR, 1), lambda i: (i, 0)),
                  pl.BlockSpec((D, D), lambda i: (0, 0)),
                  pl.BlockSpec((D, 1), lambda i: (0, 0)),
                  pl.BlockSpec((1, 1), lambda i: (0, 0))],
        out_specs=[pl.BlockSpec((NBR + PADR, 32), lambda i: (i, 0)),
                   pl.BlockSpec((NBR + PADR, 32), lambda i: (i, 0)),
                   pl.BlockSpec((NBR, 1), lambda i: (i, 0))],
        out_shape=[jax.ShapeDtypeStruct((N_P, 32), f32),
                   jax.ShapeDtypeStruct((N_P, 32), f32),
                   jax.ShapeDtypeStruct((N, 1), f32)],
    )(nma, nmb, hv, wg, wh, bh)


def _tc_e(sma, smb, wha, whb, wn):
    return pl.pallas_call(
        _e_body,
        grid=(EBLK,),
        in_specs=[pl.BlockSpec((NHB, 32), lambda i: (i, 0)),
                  pl.BlockSpec((NHB, 32), lambda i: (i, 0)),
                  pl.BlockSpec((D, D), lambda i: (0, 0)),
                  pl.BlockSpec((D, D), lambda i: (0, 0)),
                  pl.BlockSpec((D, D), lambda i: (0, 0))],
        out_specs=[pl.BlockSpec((NHB, 32), lambda i: (i, 0))] * 2,
        out_shape=[jax.ShapeDtypeStruct((NHP, 32), f32)] * 2,
    )(sma, smb, wha, whb, wn)


def _tc_e3(sma, smb, wha, whb, wh, bh):
    return pl.pallas_call(
        _e3_body,
        grid=(EBLK,),
        in_specs=[pl.BlockSpec((NHB, 32), lambda i: (i, 0)),
                  pl.BlockSpec((NHB, 32), lambda i: (i, 0)),
                  pl.BlockSpec((D, D), lambda i: (0, 0)),
                  pl.BlockSpec((D, D), lambda i: (0, 0)),
                  pl.BlockSpec((D, 1), lambda i: (0, 0)),
                  pl.BlockSpec((1, 1), lambda i: (0, 0))],
        out_specs=pl.BlockSpec((NHB, 1), lambda i: (i, 0)),
        out_shape=jax.ShapeDtypeStruct((NHP, 1), f32),
    )(sma, smb, wha, whb, wh, bh)


def _sc_edge_body(ta_ref, tb_ref, ewa_ref, ewb_ref, eidx_ref, dst_ref,
                  nma_ref, nmb_ref,
                  acc, gi0, gi1, di0, di1, er0, er1, rw0, rw1, zb,
                  si0, si1, sd0, sd1, se0, se1, sg0, sg1, ss0, ss1):
    c = lax.axis_index("c")
    s = lax.axis_index("s")

    def _zrow(i, _):
        zb[i, pl.ds(0, 16)] = jnp.zeros((16,), f32)
        zb[i, pl.ds(16, 16)] = jnp.zeros((16,), f32)
        return 0
    lax.fori_loop(0, 125, _zrow, 0)

    def _zcp(j, _):
        pltpu.sync_copy(zb, acc.at[pl.ds(s * 3125 + j * 125, 125), :])
        return 0
    lax.fori_loop(0, 25, _zcp, 0)
    plsc.subcore_barrier()

    def _run(t_ref, ew_ref):
        base = s * EPT
        gi = (gi0, gi1)
        di = (di0, di1)
        er = (er0, er1)
        rw = (rw0, rw1)
        si = (si0, si1)
        sd = (sd0, sd1)
        se = (se0, se1)
        sg = (sg0, sg1)
        ss = (ss0, ss1)

        def start_inputs(b, p):
            e0 = base + b * EB
            pltpu.async_copy(eidx_ref.at[pl.ds(e0, EB)], gi[p], si[p])
            pltpu.async_copy(dst_ref.at[pl.ds(e0, EB)], di[p], sd[p])
            pltpu.async_copy(ew_ref.at[pl.ds(e0, EB), :], er[p], se[p])

        def wait_idx(p):
            pltpu.make_async_copy(eidx_ref.at[pl.ds(0, EB)], gi[p], si[p]).wait()

        def wait_dst(p):
            pltpu.make_async_copy(dst_ref.at[pl.ds(0, EB)], di[p], sd[p]).wait()

        def wait_ew(p):
            pltpu.make_async_copy(ew_ref.at[pl.ds(0, EB), :], er[p], se[p]).wait()

        def start_gather(p):
            pltpu.async_copy(t_ref.at[gi[p]], rw[p], sg[p])

        def wait_gather(p):
            pltpu.make_async_copy(t_ref.at[gi[p]], rw[p], sg[p]).wait()

        def start_scatter(p):
            pltpu.async_copy(rw[p], acc.at[di[p]], ss[p], add=True)

        def wait_scatter(p):
            pltpu.make_async_copy(rw[p], acc.at[di[p]], ss[p]).wait()

        def multiply(p):
            rb, eb = rw[p], er[p]

            def _m(i, _):
                rb[i, pl.ds(0, 16)] = rb[i, pl.ds(0, 16)] * eb[i, pl.ds(0, 16)]
                rb[i, pl.ds(16, 16)] = rb[i, pl.ds(16, 16)] * eb[i, pl.ds(16, 16)]
                return 0
            lax.fori_loop(0, EB, _m, 0)

        start_inputs(0, 0)
        wait_idx(0)
        start_gather(0)

        def pair(t, _):
            b1 = 2 * t + 1
            b2 = 2 * t + 2

            @pl.when(b1 < NB_E)
            def _():
                @pl.when(t > 0)
                def _():
                    wait_scatter(1)
                start_inputs(b1, 1)

            wait_ew(0)
            wait_dst(0)
            wait_gather(0)
            multiply(0)
            start_scatter(0)

            @pl.when(b1 < NB_E)
            def _():
                @pl.when(b2 < NB_E)
                def _():
                    wait_scatter(0)
                    start_inputs(b2, 0)
                wait_idx(1)
                start_gather(1)
                wait_ew(1)
                wait_dst(1)
                wait_gather(1)
                multiply(1)
                start_scatter(1)

                @pl.when(b2 < NB_E)
                def _():
                    wait_idx(0)
                    start_gather(0)
            return 0

        lax.fori_loop(0, NPAIR_E, pair, 0)
        wait_scatter(0)
        wait_scatter(1)

    @pl.when(c == 0)
    def _():
        _run(ta_ref, ewa_ref)

    @pl.when(c == 1)
    def _():
        _run(tb_ref, ewb_ref)

    plsc.subcore_barrier()

    def _wout(nm_ref):
        @pl.when(s < NS - 1)
        def _():
            pltpu.sync_copy(acc.at[pl.ds(s * 3128, 3128), :],
                            nm_ref.at[pl.ds(s * 3128, 3128), :])

        @pl.when(s == NS - 1)
        def _():
            pltpu.sync_copy(acc.at[pl.ds(15 * 3128, 3080), :],
                            nm_ref.at[pl.ds(15 * 3128, 3080), :])

    @pl.when(c == 0)
    def _():
        _wout(nma_ref)

    @pl.when(c == 1)
    def _():
        _wout(nmb_ref)


def _sc_edge(ta, tb, ewa, ewb, eidx, dst):
    mesh = plsc.VectorSubcoreMesh(core_axis_name="c", subcore_axis_name="s")
    fn = pl.kernel(
        _sc_edge_body,
        out_type=[jax.ShapeDtypeStruct((N, 32), f32)] * 2,
        mesh=mesh,
        compiler_params=pltpu.CompilerParams(use_tc_tiling_on_sc=False, needs_layout_passes=False),
        scratch_types=[
            pltpu.VMEM_SHARED((N, 32), f32),
            pltpu.VMEM((EB,), jnp.int32), pltpu.VMEM((EB,), jnp.int32),
            pltpu.VMEM((EB,), jnp.int32), pltpu.VMEM((EB,), jnp.int32),
            pltpu.VMEM((EB, 32), f32), pltpu.VMEM((EB, 32), f32),
            pltpu.VMEM((EB, 32), f32), pltpu.VMEM((EB, 32), f32),
            pltpu.VMEM((125, 32), f32),
        ] + [pltpu.SemaphoreType.DMA] * 10,
    )
    return fn(ta, tb, ewa, ewb, eidx, dst)


def _sc_seg_body(xaa_ref, xab_ref, seg_ref, sma_ref, smb_ref,
                 acc, si_0, si_1, xr0, xr1, zb,
                 s_i0, s_i1, s_x0, s_x1, s_s0, s_s1):
    c = lax.axis_index("c")
    s = lax.axis_index("s")

    def _zrow(i, _):
        zb[i, pl.ds(0, 16)] = jnp.zeros((16,), f32)
        zb[i, pl.ds(16, 16)] = jnp.zeros((16,), f32)
        return 0
    lax.fori_loop(0, 125, _zrow, 0)

    def _zcp(j, _):
        pltpu.sync_copy(zb.at[pl.ds(0, 121), :], acc.at[pl.ds(s * 1563 + j * 121, 121), :])
        return 0
    lax.fori_loop(0, 12, _zcp, 0)

    pltpu.sync_copy(zb.at[pl.ds(0, 111), :], acc.at[pl.ds(s * 1563 + 1452, 111), :])
    plsc.subcore_barrier()

    def _run(xa_ref):
        si = (si_0, si_1)
        xr = (xr0, xr1)
        ssi = (s_i0, s_i1)
        ssx = (s_x0, s_x1)
        sss = (s_s0, s_s1)

        def start_inputs(b, p):
            r0 = b * EB
            pltpu.async_copy(seg_ref.at[pl.ds(r0, EB)], si[p], ssi[p])
            pltpu.async_copy(xa_ref.at[pl.ds(r0, EB), :], xr[p], ssx[p])

        def wait_inputs(p):
            pltpu.make_async_copy(seg_ref.at[pl.ds(0, EB)], si[p], ssi[p]).wait()
            pltpu.make_async_copy(xa_ref.at[pl.ds(0, EB), :], xr[p], ssx[p]).wait()

        def start_scatter(p):
            pltpu.async_copy(xr[p], acc.at[si[p]], sss[p], add=True)

        def wait_scatter(p):
            pltpu.make_async_copy(xr[p], acc.at[si[p]], sss[p]).wait()

        @pl.when(s < NB_N)
        def _():
            start_inputs(s, 0)

        def pair(t, _):
            j0 = 2 * t
            j1 = 2 * t + 1
            b0 = s + 16 * j0
            b1 = s + 16 * j1
            b2 = s + 16 * (j1 + 1)

            @pl.when(b0 < NB_N)
            def _():
                @pl.when(b1 < NB_N)
                def _():
                    @pl.when(t > 0)
                    def _():
                        wait_scatter(1)
                    start_inputs(b1, 1)
                wait_inputs(0)
                start_scatter(0)

            @pl.when(b1 < NB_N)
            def _():
                @pl.when(b2 < NB_N)
                def _():
                    wait_scatter(0)
                    start_inputs(b2, 0)
                wait_inputs(1)
                start_scatter(1)
            return 0

        lax.fori_loop(0, 13, pair, 0)

        @pl.when(s < NB_N)
        def _():
            wait_scatter(0)

        @pl.when(s + 16 < NB_N)
        def _():
            wait_scatter(1)

    @pl.when(c == 0)
    def _():
        _run(xaa_ref)

    @pl.when(c == 1)
    def _():
        _run(xab_ref)

    plsc.subcore_barrier()

    def _wout(sm_ref):
        @pl.when(s < NS - 1)
        def _():
            pltpu.sync_copy(acc.at[pl.ds(s * 1568, 1568), :],
                            sm_ref.at[pl.ds(s * 1568, 1568), :])

        @pl.when(s == NS - 1)
        def _():
            pltpu.sync_copy(acc.at[pl.ds(15 * 1568, 1488), :],
                            sm_ref.at[pl.ds(15 * 1568, 1488), :])

    @pl.when(c == 0)
    def _():
        _wout(sma_ref)

    @pl.when(c == 1)
    def _():
        _wout(smb_ref)


def _sc_seg(xaa, xab, seg_p):
    mesh = plsc.VectorSubcoreMesh(core_axis_name="c", subcore_axis_name="s")
    fn = pl.kernel(
        _sc_seg_body,
        out_type=[jax.ShapeDtypeStruct((NHP, 32), f32)] * 2,
        mesh=mesh,
        compiler_params=pltpu.CompilerParams(use_tc_tiling_on_sc=False, needs_layout_passes=False),
        scratch_types=[
            pltpu.VMEM_SHARED((NHP, 32), f32),
            pltpu.VMEM((EB,), jnp.int32), pltpu.VMEM((EB,), jnp.int32),
            pltpu.VMEM((EB, 32), f32), pltpu.VMEM((EB, 32), f32),
            pltpu.VMEM((125, 32), f32),
        ] + [pltpu.SemaphoreType.DMA] * 6,
    )
    return fn(xaa, xab, seg_p)


def _sc_map_body(src_ref, m0_ref, m1_ref, o0_ref, o1_ref,
                 sv0, sv1, ov0, ov1, ss0, ss1, sg0, sg1):
    c = lax.axis_index("c")
    s = lax.axis_index("s")
    base = s * EPT

    def _run(m_ref, o_ref):
        sv = (sv0, sv1)
        ov = (ov0, ov1)
        ss = (ss0, ss1)
        sg = (sg0, sg1)

        def start_src(b, p):
            pltpu.async_copy(src_ref.at[pl.ds(base + b * EB, EB)], sv[p], ss[p])

        def wait_src(p):
            pltpu.make_async_copy(src_ref.at[pl.ds(0, EB)], sv[p], ss[p]).wait()

        def start_g(p):
            pltpu.async_copy(m_ref.at[sv[p]], ov[p], sg[p])

        def wait_g(p):
            pltpu.make_async_copy(m_ref.at[sv[p]], ov[p], sg[p]).wait()

        start_src(0, 0)
        wait_src(0)
        start_g(0)

        def pair(t, _):
            b0 = 2 * t
            b1 = 2 * t + 1
            b2 = 2 * t + 2

            @pl.when(b1 < NB_E)
            def _():
                start_src(b1, 1)
            wait_g(0)
            pltpu.sync_copy(ov[0], o_ref.at[pl.ds(base + b0 * EB, EB)])

            @pl.when(b1 < NB_E)
            def _():
                wait_src(1)
                start_g(1)

                @pl.when(b2 < NB_E)
                def _():
                    start_src(b2, 0)
                wait_g(1)
                pltpu.sync_copy(ov[1], o_ref.at[pl.ds(base + b1 * EB, EB)])

                @pl.when(b2 < NB_E)
                def _():
                    wait_src(0)
                    start_g(0)
            return 0

        lax.fori_loop(0, NPAIR_E, pair, 0)

    @pl.when(c == 0)
    def _():
        _run(m0_ref, o0_ref)

    @pl.when(c == 1)
    def _():
        _run(m1_ref, o1_ref)


def _sc_map(src_p, m0, m1):
    mesh = plsc.VectorSubcoreMesh(core_axis_name="c", subcore_axis_name="s")
    fn = pl.kernel(
        _sc_map_body,
        out_type=[jax.ShapeDtypeStruct((E_P,), jnp.int32)] * 2,
        mesh=mesh,
        compiler_params=pltpu.CompilerParams(use_tc_tiling_on_sc=False, needs_layout_passes=False),
        scratch_types=[
            pltpu.VMEM((EB,), jnp.int32), pltpu.VMEM((EB,), jnp.int32),
            pltpu.VMEM((EB,), jnp.int32), pltpu.VMEM((EB,), jnp.int32),
        ] + [pltpu.SemaphoreType.DMA] * 4,
    )
    return fn(src_p, m0, m1)


def _sc_fin_body(tf_ref, fidx_ref, out_ref, tfv, fv, ov, s_t, s_f):
    c = lax.axis_index("c")
    s = lax.axis_index("s")
    wid = c * NS + s
    pltpu.async_copy(tf_ref, tfv, s_t).wait()

    def step(j, _):
        b = wid + 32 * j

        @pl.when(b < NBF)
        def _():
            pltpu.async_copy(fidx_ref.at[pl.ds(b * EB, EB)], fv, s_f).wait()
            for k in range(8):
                idx16 = fv[pl.ds(k * 16, 16)]
                ov[pl.ds(k * 16, 16)] = plsc.load_gather(tfv, [idx16])

            @pl.when(b < NBF - 1)
            def _():
                pltpu.sync_copy(ov, out_ref.at[pl.ds(b * EB, EB)])

            @pl.when(b == NBF - 1)
            def _():
                pltpu.sync_copy(ov.at[pl.ds(0, FTAIL)],
                                out_ref.at[pl.ds(b * EB, FTAIL)])
        return 0

    lax.fori_loop(0, 13, step, 0)


def _sc_fin(tf, fidx_p):
    mesh = plsc.VectorSubcoreMesh(core_axis_name="c", subcore_axis_name="s")
    fn = pl.kernel(
        _sc_fin_body,
        out_type=jax.ShapeDtypeStruct((N,), f32),
        mesh=mesh,
        compiler_params=pltpu.CompilerParams(use_tc_tiling_on_sc=False, needs_layout_passes=False),
        scratch_types=[
            pltpu.VMEM((N + NH,), f32),
            pltpu.VMEM((EB,), jnp.int32),
            pltpu.VMEM((EB,), f32),
            pltpu.SemaphoreType.DMA, pltpu.SemaphoreType.DMA,
        ],
    )
    return fn(tf, fidx_p)


def kernel(node_idx, edge_index_no, edge_attr_no, z, canonical, embed,
           Wx, We, Wg, Wha, Whb, W_head, b_head):
    src = edge_index_no[0]
    dst = edge_index_no[1]
    heavy = z > 1
    seg = jnp.where(heavy, canonical, NH).astype(jnp.int32)
    segs = [jnp.concatenate([seg[i * NBR:(i + 1) * NBR],
                             jnp.full((PADR,), NH, jnp.int32)]) for i in range(BLK)]
    seg_p = jnp.concatenate(segs)
    hv = heavy.astype(f32)[:, None]

    ezpad = jnp.zeros((E_P - E,), jnp.int32)
    src_p = jnp.concatenate([src.astype(jnp.int32), ezpad])
    fidx = jnp.where(heavy, N + canonical, jnp.arange(N)).astype(jnp.int32)
    eidx0_p, eidx_p = _sc_map(src_p, node_idx.astype(jnp.int32), fidx)
    dst_p = jnp.concatenate([dst.astype(jnp.int32), ezpad])
    ea_p = jnp.concatenate([edge_attr_no, jnp.zeros((E_P - E, DE), f32)])

    fidx_p = jnp.concatenate([fidx, jnp.zeros((FPAD - N,), jnp.int32)])

    def _jnp_edge(ta, tb, ewa, ewb, eidx_l, dst_p):
        t48 = jnp.concatenate([ta[:, :24], tb[:, :24]], axis=1)
        ew48 = jnp.concatenate([ewa[:, :24], ewb[:, :24]], axis=1)
        nm48 = jax.ops.segment_sum(jnp.take(t48, eidx_l, axis=0) * ew48,
                                   dst_p, num_segments=N)
        zz = jnp.zeros((N, 8), f32)
        return (jnp.concatenate([nm48[:, :24], zz], axis=1),
                jnp.concatenate([nm48[:, 24:], zz], axis=1))

    def _jnp_seg(xaa, xab, seg_p):
        return (jax.ops.segment_sum(xaa, seg_p, num_segments=NHP),
                jax.ops.segment_sum(xab, seg_p, num_segments=NHP))

    def _jnp_fin(tf, fidx_p):
        return jnp.take(tf, fidx_p[:N])

    ta, tb = _tc_t0(embed, Wx[0])
    eidx_l = eidx0_p
    aout = bout = None
    for l in range(L):
        ewa, ewb = _tc_ew(ea_p, We[l])
        if _USE_SC_EDGE:
            nma, nmb = _sc_edge(ta, tb, ewa, ewb, eidx_l, dst_p)
        else:
            nma, nmb = _jnp_edge(ta, tb, ewa, ewb, eidx_l, dst_p)
        if l < L - 1:
            xaa, xab, xwa, xwb = _tc_c(nma, nmb, hv, Wg[l], Wx[l + 1])
            sma, smb = (_sc_seg if _USE_SC_SEG else _jnp_seg)(xaa, xab, seg_p)
            twa, twb = _tc_e(sma, smb, Wha[l], Whb[l], Wx[l + 1])
            ta = jnp.concatenate([xwa, twa])
            tb = jnp.concatenate([xwb, twb])
            eidx_l = eidx_p
        else:
            bh = b_head.reshape(1, 1)
            xaa, xab, aout = _tc_c3(nma, nmb, hv, Wg[l], W_head, bh)
            sma, smb = (_sc_seg if _USE_SC_SEG else _jnp_seg)(xaa, xab, seg_p)
            bout = _tc_e3(sma, smb, Wha[l], Whb[l], W_head, bh)

    tf = jnp.concatenate([aout[:, 0], bout[:NH, 0]])
    out = (_sc_fin if _USE_SC_FIN else _jnp_fin)(tf, fidx_p)
    return out[:, None]
